# baseline (speedup 1.0000x reference)
import functools

import jax
import jax.numpy as jnp
from jax import lax
from jax.experimental import pallas as pl
from jax.experimental.pallas import tpu as pltpu
from jax.experimental.pallas import tpu_sc as plsc

N = 10000
E = 160000
D_X = 256
D_LAB = 64
D_IN = 320
D_HID = 512
D_OUT = 64

NC = 2
NS = 16
EB = 128

E_PAD = 163840
ROWS_TOTAL = E_PAD // EB
RPT = ROWS_TOTAL // NS
DEG_RPT = ROWS_TOTAL // (NC * NS)
ACC_ROWS = 10240
ZROWS = ACC_ROWS // NS
ORPT = 632
ORPT_LAST = N - (NS - 1) * ORPT
DUMMY = N

_MESH = plsc.VectorSubcoreMesh(core_axis_name="c", subcore_axis_name="s")
_SC_PARAMS = pltpu.CompilerParams(use_tc_tiling_on_sc=False)
_SC_PARAMS_NL = pltpu.CompilerParams(
    use_tc_tiling_on_sc=False, needs_layout_passes=False)


L1D = 32
L1R = D_IN // (NC * L1D)
L1NB = 4
L1G = RPT // (2 * L1NB)


def _fast_rsqrt(x):
    i = plsc.bitcast(x, jnp.int32)
    i = jnp.int32(0x5F3759DF) - lax.shift_right_logical(i, 1)
    y = plsc.bitcast(i, jnp.float32)
    for _ in range(3):
        y = y * (1.5 - 0.5 * x * y * y)
    return y


@functools.partial(
    pl.kernel,
    out_type=tuple(
        [jax.ShapeDtypeStruct((N, L1D), jnp.float32)] * (NC * L1R)
        + [jax.ShapeDtypeStruct((ACC_ROWS,), jnp.float32)]),
    mesh=_MESH,
    scratch_types=[
        pltpu.VMEM((RPT, EB), jnp.int32),
        pltpu.VMEM((RPT, EB), jnp.int32),
        pltpu.VMEM((EB,), jnp.float32),
        pltpu.VMEM((ZROWS,), jnp.float32),
        pltpu.VMEM((ZROWS,), jnp.float32),
        pltpu.VMEM((ZROWS, L1D), jnp.float32),
        pltpu.VMEM((L1NB, EB, L1D), jnp.float32),
        pltpu.VMEM((L1NB, EB, L1D), jnp.float32),
        pltpu.VMEM_SHARED((ACC_ROWS,), jnp.float32),
        pltpu.VMEM_SHARED((ACC_ROWS, L1D), jnp.float32),
        pltpu.VMEM_SHARED((N, L1D), jnp.float32),
        pltpu.SemaphoreType.DMA,
        pltpu.SemaphoreType.DMA,
        pltpu.SemaphoreType.DMA,
        pltpu.SemaphoreType.DMA,
    ],
    compiler_params=_SC_PARAMS_NL,
)
def _fused_l1(*refs):
    nt = NC * L1R
    tbls = refs[:nt]
    src2d, dst2d, zeros1_hbm, zeros_hbm, ones_hbm = refs[nt:nt + 5]
    outs = refs[nt + 5:nt + 5 + nt]
    dinv_out = refs[nt + 5 + nt]
    (src_v, dst_v, ones_v, dv_v, dvt_v, stage_v, buf_a, buf_b,
     dacc, acc, tspm, gs_a, gs_b, ss_a, ss_b) = refs[nt + 6 + nt:]
    cid = lax.axis_index("c")
    tid = lax.axis_index("s")
    row0 = pl.multiple_of(tid * RPT, 8)
    z0 = pl.multiple_of(tid * ZROWS, 8)
    o0 = pl.multiple_of(tid * ORPT, 8)

    pltpu.sync_copy(src2d.at[pl.ds(row0, RPT)], src_v)
    pltpu.sync_copy(dst2d.at[pl.ds(row0, RPT)], dst_v)
    pltpu.sync_copy(ones_hbm, ones_v)
    pltpu.sync_copy(zeros1_hbm, dacc.at[pl.ds(z0, ZROWS)])
    plsc.subcore_barrier()

    def deg_body(j, carry):
        pltpu.sync_copy(ones_v, dacc.at[dst_v.at[j]], add=True)
        return carry

    lax.fori_loop(0, RPT, deg_body, 0)
    plsc.subcore_barrier()

    pltpu.sync_copy(dacc.at[pl.ds(z0, ZROWS)], dv_v)

    def rsq_body(k, carry):
        d = dv_v[pl.ds(k * 16, 16)] + 1.0
        dv_v[pl.ds(k * 16, 16)] = _fast_rsqrt(d)
        return carry

    lax.fori_loop(0, ZROWS // 16, rsq_body, 0)
    pltpu.sync_copy(dv_v, dacc.at[pl.ds(z0, ZROWS)])

    @pl.when(cid == 0)
    def _():
        pltpu.sync_copy(dv_v, dinv_out.at[pl.ds(z0, ZROWS)])

    plsc.subcore_barrier()

    def stage_scaled(tbl):
        def do(start, nrows):
            pltpu.sync_copy(tbl.at[pl.ds(start, nrows)],
                            stage_v.at[pl.ds(0, nrows)])
            pltpu.sync_copy(dacc.at[pl.ds(start, nrows)],
                            dvt_v.at[pl.ds(0, nrows)])

            def grpb(g, carry):
                s16 = dvt_v[pl.ds(g * 16, 16)]
                for i in range(16):
                    r = g * 16 + i
                    s = s16[i]
                    stage_v[r, pl.ds(0, 16)] = stage_v[r, pl.ds(0, 16)] * s
                    stage_v[r, pl.ds(16, 16)] = \
                        stage_v[r, pl.ds(16, 16)] * s
                return carry

            lax.fori_loop(0, nrows // 16, grpb, 0)
            pltpu.sync_copy(stage_v.at[pl.ds(0, nrows)],
                            tspm.at[pl.ds(start, nrows)])

        @pl.when(tid < NS - 1)
        def _():
            do(z0, ZROWS)

        @pl.when(tid == NS - 1)
        def _():
            do((NS - 1) * ZROWS, N - (NS - 1) * ZROWS)

    def gather(row, bufs, b, sem):
        return pltpu.async_copy(tspm.at[src_v.at[row]], bufs.at[b], sem)

    def process():
        for b in range(L1NB):
            gather(b, buf_a, b, gs_a)

        def body(g, carry):
            base_a = 2 * L1NB * g
            base_b = base_a + L1NB
            for b in range(L1NB):
                pltpu.make_async_copy(
                    tspm.at[src_v.at[base_a + b]], buf_a.at[b],
                    gs_a).wait()
            sca = [pltpu.async_copy(
                buf_a.at[b], acc.at[dst_v.at[base_a + b]], ss_a,
                add=True) for b in range(L1NB)]
            gb = [gather(base_b + b, buf_b, b, gs_b) for b in range(L1NB)]
            for d in sca:
                d.wait()
            row_n = jnp.where(g + 1 < L1G, base_a + 2 * L1NB, 0)
            for b in range(L1NB):
                gather(row_n + b, buf_a, b, gs_a)
            for d in gb:
                d.wait()
            scb = [pltpu.async_copy(
                buf_b.at[b], acc.at[dst_v.at[base_b + b]], ss_b,
                add=True) for b in range(L1NB)]
            for d in scb:
                d.wait()
            return carry

        lax.fori_loop(0, L1G, body, 0)
        for b in range(L1NB):
            pltpu.make_async_copy(
                tspm.at[src_v.at[b]], buf_a.at[b], gs_a).wait()

    def copy_out(out):
        @pl.when(tid < NS - 1)
        def _():
            pltpu.sync_copy(acc.at[pl.ds(o0, ORPT)], out.at[pl.ds(o0, ORPT)])

        @pl.when(tid == NS - 1)
        def _():
            pltpu.sync_copy(acc.at[pl.ds((NS - 1) * ORPT, ORPT_LAST)],
                            out.at[pl.ds((NS - 1) * ORPT, ORPT_LAST)])

    for r in range(L1R):
        @pl.when(cid == 0)
        def _(r=r):
            stage_scaled(tbls[r])

        @pl.when(cid == 1)
        def _(r=r):
            stage_scaled(tbls[L1R + r])

        pltpu.sync_copy(zeros_hbm, acc.at[pl.ds(z0, ZROWS)])
        plsc.subcore_barrier()
        process()
        plsc.subcore_barrier()

        @pl.when(cid == 0)
        def _(r=r):
            copy_out(outs[r])

        @pl.when(cid == 1)
        def _(r=r):
            copy_out(outs[L1R + r])

        plsc.subcore_barrier()


def _make_agg_kernel(d_chunk, n_rounds, nbuf, spmem_table=False):
    nt = NC * n_rounds
    ngrp = RPT // (2 * nbuf)

    @functools.partial(
        pl.kernel,
        out_type=tuple(
            jax.ShapeDtypeStruct((N, d_chunk), jnp.float32)
            for _ in range(nt)),
        mesh=_MESH,
        scratch_types=[
            pltpu.VMEM((RPT, EB), jnp.int32),
            pltpu.VMEM((RPT, EB), jnp.int32),
            pltpu.VMEM((nbuf, EB, d_chunk), jnp.float32),
            pltpu.VMEM((nbuf, EB, d_chunk), jnp.float32),
            pltpu.VMEM_SHARED((ACC_ROWS, d_chunk), jnp.float32),
            (pltpu.VMEM_SHARED((N, d_chunk), jnp.float32)
             if spmem_table else pltpu.VMEM((8,), jnp.float32)),
            pltpu.SemaphoreType.DMA,
            pltpu.SemaphoreType.DMA,
            pltpu.SemaphoreType.DMA,
            pltpu.SemaphoreType.DMA,
        ],
        compiler_params=_SC_PARAMS,
    )
    def agg_kernel(*refs):
        tbls = refs[:nt]
        src2d, dst2d, zeros_hbm = refs[nt:nt + 3]
        outs = refs[nt + 3:nt + 3 + nt]
        src_v, dst_v, buf_a, buf_b, acc, tspm, gs_a, gs_b, ss_a, ss_b = \
            refs[nt + 3 + nt:]
        cid = lax.axis_index("c")
        tid = lax.axis_index("s")
        row0 = pl.multiple_of(tid * RPT, 8)
        pltpu.sync_copy(src2d.at[pl.ds(row0, RPT)], src_v)
        pltpu.sync_copy(dst2d.at[pl.ds(row0, RPT)], dst_v)
        z0 = pl.multiple_of(tid * ZROWS, 8)
        o0 = pl.multiple_of(tid * ORPT, 8)

        def process(tbl):
            if spmem_table:
                @pl.when(tid < NS - 1)
                def _():
                    pltpu.sync_copy(tbl.at[pl.ds(o0, ORPT)],
                                    tspm.at[pl.ds(o0, ORPT)])

                @pl.when(tid == NS - 1)
                def _():
                    pltpu.sync_copy(
                        tbl.at[pl.ds((NS - 1) * ORPT, ORPT_LAST)],
                        tspm.at[pl.ds((NS - 1) * ORPT, ORPT_LAST)])

                plsc.subcore_barrier()
                src_tbl = tspm
            else:
                src_tbl = tbl

            def gather(row, bufs, b, sem):
                return pltpu.async_copy(
                    src_tbl.at[src_v.at[row]], bufs.at[b], sem)

            for b in range(nbuf):
                gather(b, buf_a, b, gs_a)

            def body(g, carry):
                base_a = 2 * nbuf * g
                base_b = base_a + nbuf
                for b in range(nbuf):
                    pltpu.make_async_copy(
                        src_tbl.at[src_v.at[base_a + b]], buf_a.at[b],
                        gs_a).wait()
                sca = [pltpu.async_copy(
                    buf_a.at[b], acc.at[dst_v.at[base_a + b]], ss_a,
                    add=True) for b in range(nbuf)]
                gb = [gather(base_b + b, buf_b, b, gs_b)
                      for b in range(nbuf)]
                for d in sca:
                    d.wait()
                row_n = jnp.where(g + 1 < ngrp, base_a + 2 * nbuf, 0)
                for b in range(nbuf):
                    gather(row_n + b, buf_a, b, gs_a)
                for d in gb:
                    d.wait()
                scb = [pltpu.async_copy(
                    buf_b.at[b], acc.at[dst_v.at[base_b + b]], ss_b,
                    add=True) for b in range(nbuf)]
                for d in scb:
                    d.wait()
                return carry

            lax.fori_loop(0, ngrp, body, 0)
            for b in range(nbuf):
                pltpu.make_async_copy(
                    src_tbl.at[src_v.at[b]], buf_a.at[b], gs_a).wait()

        def copy_out(out):
            @pl.when(tid < NS - 1)
            def _():
                pltpu.sync_copy(acc.at[pl.ds(o0, ORPT)],
                                out.at[pl.ds(o0, ORPT)])

            @pl.when(tid == NS - 1)
            def _():
                pltpu.sync_copy(
                    acc.at[pl.ds((NS - 1) * ORPT, ORPT_LAST)],
                    out.at[pl.ds((NS - 1) * ORPT, ORPT_LAST)])

        for r in range(n_rounds):
            pltpu.sync_copy(zeros_hbm, acc.at[pl.ds(z0, ZROWS)])
            plsc.subcore_barrier()

            @pl.when(cid == 0)
            def _(r=r):
                process(tbls[r])

            @pl.when(cid == 1)
            def _(r=r):
                process(tbls[n_rounds + r])

            plsc.subcore_barrier()

            @pl.when(cid == 0)
            def _(r=r):
                copy_out(outs[r])

            @pl.when(cid == 1)
            def _(r=r):
                copy_out(outs[n_rounds + r])

            plsc.subcore_barrier()

    return agg_kernel


_agg32 = _make_agg_kernel(32, 1, 8, spmem_table=True)


BLK_M = 400


def _mlp_body(*refs):
    a_refs = refs[:NC * L1R]
    x_ref, lab_ref, dinv_ref, w1_ref, b1_ref, w2_ref, h0_ref, h1_ref = \
        refs[NC * L1R:]
    dinv = dinv_ref[...]
    xc = jnp.concatenate([x_ref[...], lab_ref[...]], axis=1)
    agg = jnp.concatenate([a[...] for a in a_refs], axis=1)
    t = (agg + xc * dinv) * dinv
    y = jnp.dot(t, w1_ref[...], preferred_element_type=jnp.float32)
    y = jnp.maximum(y + b1_ref[...], 0.0)
    h2 = jnp.dot(y, w2_ref[...], preferred_element_type=jnp.float32)
    h2s = h2 * dinv
    h0_ref[...] = h2s[:, :32]
    h1_ref[...] = h2s[:, 32:]


_mlp = pl.pallas_call(
    _mlp_body,
    grid=(N // BLK_M,),
    in_specs=(
        [pl.BlockSpec((BLK_M, L1D), lambda i: (i, 0))] * (NC * L1R)
        + [
            pl.BlockSpec((BLK_M, D_X), lambda i: (i, 0)),
            pl.BlockSpec((BLK_M, D_LAB), lambda i: (i, 0)),
            pl.BlockSpec((BLK_M, 1), lambda i: (i, 0)),
            pl.BlockSpec((D_IN, D_HID), lambda i: (0, 0)),
            pl.BlockSpec((1, D_HID), lambda i: (0, 0)),
            pl.BlockSpec((D_HID, D_OUT), lambda i: (0, 0)),
        ]
    ),
    out_specs=[pl.BlockSpec((BLK_M, 32), lambda i: (i, 0))] * 2,
    out_shape=[jax.ShapeDtypeStruct((N, 32), jnp.float32)] * 2,
)


def _final_body(a0_ref, a1_ref, h0_ref, h1_ref, dinv_ref, b2_ref, out_ref):
    dinv = dinv_ref[...]
    t2 = jnp.concatenate(
        [a0_ref[...] + h0_ref[...], a1_ref[...] + h1_ref[...]], axis=1)
    t2 = t2 * dinv + b2_ref[...]
    m = jnp.max(t2, axis=1, keepdims=True)
    s = t2 - m
    out_ref[...] = s - jnp.log(jnp.sum(jnp.exp(s), axis=1, keepdims=True))


_final = pl.pallas_call(
    _final_body,
    grid=(N // BLK_M,),
    in_specs=(
        [pl.BlockSpec((BLK_M, 32), lambda i: (i, 0))] * 4
        + [
            pl.BlockSpec((BLK_M, 1), lambda i: (i, 0)),
            pl.BlockSpec((1, D_OUT), lambda i: (0, 0)),
        ]
    ),
    out_specs=pl.BlockSpec((BLK_M, D_OUT), lambda i: (i, 0)),
    out_shape=jax.ShapeDtypeStruct((N, D_OUT), jnp.float32),
)


def kernel(x, edge_index, label_feat, W1, b1, W2, b2):
    edge_index = edge_index.astype(jnp.int32)
    src = edge_index[0]
    dst = edge_index[1]
    pad = E_PAD - E
    src2d = jnp.concatenate(
        [src, jnp.zeros((pad,), jnp.int32)]).reshape(ROWS_TOTAL, EB)
    dst2d = jnp.concatenate(
        [dst, jnp.full((pad,), DUMMY, jnp.int32)]).reshape(ROWS_TOTAL, EB)

    ones_eb = jnp.ones((EB,), jnp.float32)
    zeros_1d = jnp.zeros((ZROWS,), jnp.float32)
    zeros_32 = jnp.zeros((ZROWS, L1D), jnp.float32)

    qs = [x[:, 32 * k:32 * (k + 1)] for k in range(8)] \
        + [label_feat[:, :32], label_feat[:, 32:]]
    fused = _fused_l1(*qs, src2d, dst2d, zeros_1d, zeros_32, ones_eb)
    aggs, dinv_flat = fused[:NC * L1R], fused[NC * L1R]
    dinv2d = dinv_flat[:N].reshape(N, 1)
    h2s0, h2s1 = _mlp(*aggs, x, label_feat, dinv2d,
                      W1, b1.reshape(1, D_HID), W2)
    g0, g1 = _agg32(h2s0, h2s1, src2d, dst2d, zeros_32)
    return _final(g0, g1, h2s0, h2s1, dinv2d, b2.reshape(1, D_OUT))


# speedup vs baseline: 1.0152x; 1.0152x over previous
import functools

import jax
import jax.numpy as jnp
from jax import lax
from jax.experimental import pallas as pl
from jax.experimental.pallas import tpu as pltpu
from jax.experimental.pallas import tpu_sc as plsc

N = 10000
E = 160000
D_X = 256
D_LAB = 64
D_IN = 320
D_HID = 512
D_OUT = 64

NC = 2
NS = 16
EB = 128

E_PAD = 163840
ROWS_TOTAL = E_PAD // EB
RPT = ROWS_TOTAL // NS
DEG_RPT = ROWS_TOTAL // (NC * NS)
ACC_ROWS = 10240
ZROWS = ACC_ROWS // NS
ORPT = 632
ORPT_LAST = N - (NS - 1) * ORPT
DUMMY = N

_MESH = plsc.VectorSubcoreMesh(core_axis_name="c", subcore_axis_name="s")
_SC_PARAMS = pltpu.CompilerParams(use_tc_tiling_on_sc=False)


@functools.partial(
    pl.kernel,
    out_type=jax.ShapeDtypeStruct((NC * ACC_ROWS,), jnp.float32),
    mesh=_MESH,
    scratch_types=[
        pltpu.VMEM((DEG_RPT, EB), jnp.int32),
        pltpu.VMEM((EB,), jnp.float32),
        pltpu.VMEM_SHARED((ACC_ROWS,), jnp.float32),
    ],
    compiler_params=_SC_PARAMS,
)
def _deg_kernel(dst2d, ones_hbm, zeros_hbm, out, dst_v, ones_v, acc):
    cid = lax.axis_index("c")
    tid = lax.axis_index("s")
    z0 = pl.multiple_of(tid * ZROWS, 8)
    pltpu.sync_copy(zeros_hbm, acc.at[pl.ds(z0, ZROWS)])
    pltpu.sync_copy(ones_hbm, ones_v)
    row0 = pl.multiple_of(cid * (NS * DEG_RPT) + tid * DEG_RPT, 8)
    pltpu.sync_copy(dst2d.at[pl.ds(row0, DEG_RPT)], dst_v)
    plsc.subcore_barrier()

    def body(j, carry):
        pltpu.sync_copy(ones_v, acc.at[dst_v.at[j]], add=True)
        return carry

    lax.fori_loop(0, DEG_RPT, body, 0)
    plsc.subcore_barrier()
    o0 = pl.multiple_of(cid * ACC_ROWS + tid * ZROWS, 8)
    pltpu.sync_copy(acc.at[pl.ds(z0, ZROWS)], out.at[pl.ds(o0, ZROWS)])


def _make_agg_kernel(d_chunk, n_rounds, nbuf, spmem_table=False):
    nt = NC * n_rounds
    ngrp = RPT // (2 * nbuf)

    @functools.partial(
        pl.kernel,
        out_type=tuple(
            jax.ShapeDtypeStruct((N, d_chunk), jnp.float32)
            for _ in range(nt)),
        mesh=_MESH,
        scratch_types=[
            pltpu.VMEM((RPT, EB), jnp.int32),
            pltpu.VMEM((RPT, EB), jnp.int32),
            pltpu.VMEM((nbuf, EB, d_chunk), jnp.float32),
            pltpu.VMEM((nbuf, EB, d_chunk), jnp.float32),
            pltpu.VMEM_SHARED((ACC_ROWS, d_chunk), jnp.float32),
            (pltpu.VMEM_SHARED((N, d_chunk), jnp.float32)
             if spmem_table else pltpu.VMEM((8,), jnp.float32)),
            pltpu.SemaphoreType.DMA,
            pltpu.SemaphoreType.DMA,
            pltpu.SemaphoreType.DMA,
            pltpu.SemaphoreType.DMA,
        ],
        compiler_params=_SC_PARAMS,
    )
    def agg_kernel(*refs):
        tbls = refs[:nt]
        src2d, dst2d, zeros_hbm = refs[nt:nt + 3]
        outs = refs[nt + 3:nt + 3 + nt]
        src_v, dst_v, buf_a, buf_b, acc, tspm, gs_a, gs_b, ss_a, ss_b = \
            refs[nt + 3 + nt:]
        cid = lax.axis_index("c")
        tid = lax.axis_index("s")
        row0 = pl.multiple_of(tid * RPT, 8)
        pltpu.sync_copy(src2d.at[pl.ds(row0, RPT)], src_v)
        pltpu.sync_copy(dst2d.at[pl.ds(row0, RPT)], dst_v)
        z0 = pl.multiple_of(tid * ZROWS, 8)
        o0 = pl.multiple_of(tid * ORPT, 8)

        def process(tbl):
            if spmem_table:
                @pl.when(tid < NS - 1)
                def _():
                    pltpu.sync_copy(tbl.at[pl.ds(o0, ORPT)],
                                    tspm.at[pl.ds(o0, ORPT)])

                @pl.when(tid == NS - 1)
                def _():
                    pltpu.sync_copy(
                        tbl.at[pl.ds((NS - 1) * ORPT, ORPT_LAST)],
                        tspm.at[pl.ds((NS - 1) * ORPT, ORPT_LAST)])

                plsc.subcore_barrier()
                src_tbl = tspm
            else:
                src_tbl = tbl

            def gather(row, bufs, b, sem):
                return pltpu.async_copy(
                    src_tbl.at[src_v.at[row]], bufs.at[b], sem)

            for b in range(nbuf):
                gather(b, buf_a, b, gs_a)

            def body(g, carry):
                base_a = 2 * nbuf * g
                base_b = base_a + nbuf
                for b in range(nbuf):
                    pltpu.make_async_copy(
                        src_tbl.at[src_v.at[base_a + b]], buf_a.at[b],
                        gs_a).wait()
                sca = [pltpu.async_copy(
                    buf_a.at[b], acc.at[dst_v.at[base_a + b]], ss_a,
                    add=True) for b in range(nbuf)]
                gb = [gather(base_b + b, buf_b, b, gs_b)
                      for b in range(nbuf)]
                for d in sca:
                    d.wait()
                row_n = jnp.where(g + 1 < ngrp, base_a + 2 * nbuf, 0)
                for b in range(nbuf):
                    gather(row_n + b, buf_a, b, gs_a)
                for d in gb:
                    d.wait()
                scb = [pltpu.async_copy(
                    buf_b.at[b], acc.at[dst_v.at[base_b + b]], ss_b,
                    add=True) for b in range(nbuf)]
                for d in scb:
                    d.wait()
                return carry

            lax.fori_loop(0, ngrp, body, 0)
            for b in range(nbuf):
                pltpu.make_async_copy(
                    src_tbl.at[src_v.at[b]], buf_a.at[b], gs_a).wait()

        def copy_out(out):
            @pl.when(tid < NS - 1)
            def _():
                pltpu.sync_copy(acc.at[pl.ds(o0, ORPT)],
                                out.at[pl.ds(o0, ORPT)])

            @pl.when(tid == NS - 1)
            def _():
                pltpu.sync_copy(
                    acc.at[pl.ds((NS - 1) * ORPT, ORPT_LAST)],
                    out.at[pl.ds((NS - 1) * ORPT, ORPT_LAST)])

        for r in range(n_rounds):
            pltpu.sync_copy(zeros_hbm, acc.at[pl.ds(z0, ZROWS)])
            plsc.subcore_barrier()

            @pl.when(cid == 0)
            def _(r=r):
                process(tbls[r])

            @pl.when(cid == 1)
            def _(r=r):
                process(tbls[n_rounds + r])

            plsc.subcore_barrier()

            @pl.when(cid == 0)
            def _(r=r):
                copy_out(outs[r])

            @pl.when(cid == 1)
            def _(r=r):
                copy_out(outs[n_rounds + r])

            plsc.subcore_barrier()

    return agg_kernel


_agg40 = _make_agg_kernel(40, 4, 5, spmem_table=True)
_agg32 = _make_agg_kernel(32, 1, 8, spmem_table=True)


def _dinv_of(deg_ref):
    deg = deg_ref[:, 0] + deg_ref[:, 1] + 1.0
    return lax.rsqrt(deg)[:, None]


BLK_P = 2000


def _prescale_body(x_ref, lab_ref, deg_ref, *q_refs):
    dinv = _dinv_of(deg_ref)
    xc = jnp.concatenate([x_ref[...], lab_ref[...]], axis=1) * dinv
    for k in range(8):
        q_refs[k][...] = xc[:, 40 * k:40 * (k + 1)]


_prescale = pl.pallas_call(
    _prescale_body,
    grid=(N // BLK_P,),
    in_specs=[
        pl.BlockSpec((BLK_P, D_X), lambda i: (i, 0)),
        pl.BlockSpec((BLK_P, D_LAB), lambda i: (i, 0)),
        pl.BlockSpec((BLK_P, 2), lambda i: (i, 0)),
    ],
    out_specs=[pl.BlockSpec((BLK_P, 40), lambda i: (i, 0))] * 8,
    out_shape=[jax.ShapeDtypeStruct((N, 40), jnp.float32)] * 8,
)

BLK_M = 400


def _mlp_body(*refs):
    a_refs = refs[:8]
    q_refs = refs[8:16]
    deg_ref, w1_ref, b1_ref, w2_ref, h0_ref, h1_ref = refs[16:]
    dinv = _dinv_of(deg_ref)
    t = jnp.concatenate(
        [(a_refs[k][...] + q_refs[k][...]) for k in range(8)],
        axis=1) * dinv
    y = jnp.dot(t, w1_ref[...], preferred_element_type=jnp.float32)
    y = jnp.maximum(y + b1_ref[...], 0.0)
    h2 = jnp.dot(y, w2_ref[...], preferred_element_type=jnp.float32)
    h2s = h2 * dinv
    h0_ref[...] = h2s[:, :32]
    h1_ref[...] = h2s[:, 32:]


_mlp = pl.pallas_call(
    _mlp_body,
    grid=(N // BLK_M,),
    in_specs=(
        [pl.BlockSpec((BLK_M, 40), lambda i: (i, 0))] * 16
        + [
            pl.BlockSpec((BLK_M, 2), lambda i: (i, 0)),
            pl.BlockSpec((D_IN, D_HID), lambda i: (0, 0)),
            pl.BlockSpec((1, D_HID), lambda i: (0, 0)),
            pl.BlockSpec((D_HID, D_OUT), lambda i: (0, 0)),
        ]
    ),
    out_specs=[pl.BlockSpec((BLK_M, 32), lambda i: (i, 0))] * 2,
    out_shape=[jax.ShapeDtypeStruct((N, 32), jnp.float32)] * 2,
)


def _final_body(a0_ref, a1_ref, h0_ref, h1_ref, deg_ref, b2_ref, out_ref):
    dinv = _dinv_of(deg_ref)
    t2 = jnp.concatenate(
        [a0_ref[...] + h0_ref[...], a1_ref[...] + h1_ref[...]], axis=1)
    t2 = t2 * dinv + b2_ref[...]
    m = jnp.max(t2, axis=1, keepdims=True)
    s = t2 - m
    out_ref[...] = s - jnp.log(jnp.sum(jnp.exp(s), axis=1, keepdims=True))


_final = pl.pallas_call(
    _final_body,
    grid=(N // BLK_M,),
    in_specs=(
        [pl.BlockSpec((BLK_M, 32), lambda i: (i, 0))] * 4
        + [
            pl.BlockSpec((BLK_M, 2), lambda i: (i, 0)),
            pl.BlockSpec((1, D_OUT), lambda i: (0, 0)),
        ]
    ),
    out_specs=pl.BlockSpec((BLK_M, D_OUT), lambda i: (i, 0)),
    out_shape=jax.ShapeDtypeStruct((N, D_OUT), jnp.float32),
)


def kernel(x, edge_index, label_feat, W1, b1, W2, b2):
    edge_index = edge_index.astype(jnp.int32)
    src = edge_index[0]
    dst = edge_index[1]
    pad = E_PAD - E
    src2d = jnp.concatenate(
        [src, jnp.zeros((pad,), jnp.int32)]).reshape(ROWS_TOTAL, EB)
    dst2d = jnp.concatenate(
        [dst, jnp.full((pad,), DUMMY, jnp.int32)]).reshape(ROWS_TOTAL, EB)

    ones_eb = jnp.ones((EB,), jnp.float32)
    zeros_1d = jnp.zeros((ZROWS,), jnp.float32)
    zeros_40 = jnp.zeros((ZROWS, 40), jnp.float32)
    zeros_32 = jnp.zeros((ZROWS, 32), jnp.float32)

    degp = _deg_kernel(dst2d, ones_eb, zeros_1d).reshape(NC, ACC_ROWS).T
    qs = _prescale(x, label_feat, degp)
    aggs = _agg40(*qs, src2d, dst2d, zeros_40)
    h2s0, h2s1 = _mlp(*aggs, *qs, degp,
                      W1, b1.reshape(1, D_HID), W2)
    g0, g1 = _agg32(h2s0, h2s1, src2d, dst2d, zeros_32)
    return _final(g0, g1, h2s0, h2s1, degp, b2.reshape(1, D_OUT))
